# baseline (speedup 1.0000x reference)
import functools

import jax
import jax.numpy as jnp
from jax.experimental import pallas as pl
from jax.experimental.pallas import tpu as pltpu

_F32 = jnp.float32


def _blk(n, target):
    if n <= target:
        return n
    for b in range(target, 0, -1):
        if n % b == 0 and b % 8 == 0:
            return b
    return n


def _gconv1_kernel(nhid, nfeat, a_ref, x_ref, msk_ref, bt_ref, w_ref,
                   wp_ref, pa_ref, h_ref, u_ref, racc_ref):
    y = jnp.dot(a_ref[...], x_ref[...], preferred_element_type=_F32)
    w = w_ref[...]
    x = jnp.concatenate(
        [jnp.dot(y[:, :nfeat], w, preferred_element_type=_F32),
         jnp.dot(y[:, nfeat:], w, preferred_element_type=_F32)], axis=1)
    x = x + bt_ref[...]
    a = pa_ref[0, 0]
    h = jnp.where(x > 0, x, a * x)
    h_ref[...] = h
    u_ref[...] = jnp.dot(h[:, :nhid], wp_ref[...], preferred_element_type=_F32)
    m = msk_ref[0]
    racc_ref[...] = jnp.dot(m, h, preferred_element_type=_F32)[:, :nhid][None]


def _gconv2a_kernel(a_ref, u_ref, p_ref):
    p_ref[...] = jnp.dot(a_ref[...], u_ref[...], preferred_element_type=_F32)


def _gconv2b_kernel(nhid, d_ref, u_ref, p_ref, ha_ref, hd_ref, ra_ref,
                    rd_ref, wb_ref, b24_ref, bb_ref, inv_ref,
                    out_ref, sc_ref):
    y = (p_ref[...]
         + jnp.dot(d_ref[...], u_ref[...], preferred_element_type=_F32)
         + b24_ref[...])
    mx = jnp.max(y, axis=1, keepdims=True)
    z = y - mx
    out_ref[...] = z - jnp.log(jnp.sum(jnp.exp(z), axis=1, keepdims=True))

    c1 = jax.nn.sigmoid(jnp.sum(ra_ref[...], axis=0) * inv_ref[0, 0])
    c2 = jax.nn.sigmoid(jnp.sum(rd_ref[...], axis=0) * inv_ref[0, 0])
    wb = wb_ref[...]
    dn = (((1,), (1,)), ((), ()))
    v1 = jax.lax.dot_general(wb, c1, dn, preferred_element_type=_F32)
    v2 = jax.lax.dot_general(wb, c2, dn, preferred_element_type=_F32)
    h1 = ha_ref[:, :nhid]
    h3 = ha_ref[:, nhid:]
    h2 = hd_ref[:, :nhid]
    h4 = hd_ref[:, nhid:]
    t1 = jnp.dot(h2, v1, preferred_element_type=_F32)
    t2 = jnp.dot(h1, v2, preferred_element_type=_F32)
    t3 = jnp.dot(h4, v1, preferred_element_type=_F32)
    t4 = jnp.dot(h3, v2, preferred_element_type=_F32)
    sc_ref[...] = jnp.concatenate([t1, t2, t3, t4], axis=1) + bb_ref[0, 0]


def kernel(feature, adj, diff, shuf_fts, sparse, msk, samp_bias1, samp_bias2,
           W1, b1, W2, b2, W3, b3, W4, b4, Wb, bb, prelu_a):
    del sparse, samp_bias1, samp_bias2
    n, nfeat = feature.shape
    nhid = W1.shape[1]
    ncls = W2.shape[1]

    bm = _blk(n, 400)
    ni = n // bm

    b1t = jnp.concatenate([b1, b1]).reshape(1, 2 * nhid)
    b3t = jnp.concatenate([b3, b3]).reshape(1, 2 * nhid)
    b24 = (b2 + b4).reshape(1, ncls)
    wb0 = Wb[0]
    bb2 = bb.reshape(1, 1)
    pa2 = prelu_a.reshape(1, 1)
    msk3 = msk.reshape(ni, 1, bm)
    inv = (1.0 / (n * jnp.sum(msk))).reshape(1, 1).astype(_F32)

    par = pltpu.CompilerParams(dimension_semantics=("parallel",),
                               vmem_limit_bytes=112 * 1024 * 1024)

    xcat = jnp.concatenate([feature, shuf_fts], axis=1)

    def gconv1(mat, w, bt, wp):
        return pl.pallas_call(
            functools.partial(_gconv1_kernel, nhid, nfeat),
            grid=(ni,),
            in_specs=[
                pl.BlockSpec((bm, n), lambda i: (i, 0)),
                pl.BlockSpec((n, 2 * nfeat), lambda i: (0, 0)),
                pl.BlockSpec((1, 1, bm), lambda i: (i, 0, 0)),
                pl.BlockSpec((1, 2 * nhid), lambda i: (0, 0)),
                pl.BlockSpec((nfeat, nhid), lambda i: (0, 0)),
                pl.BlockSpec((nhid, ncls), lambda i: (0, 0)),
                pl.BlockSpec((1, 1), lambda i: (0, 0)),
            ],
            out_specs=[
                pl.BlockSpec((bm, 2 * nhid), lambda i: (i, 0)),
                pl.BlockSpec((bm, ncls), lambda i: (i, 0)),
                pl.BlockSpec((1, 1, nhid), lambda i: (i, 0, 0)),
            ],
            out_shape=[
                jax.ShapeDtypeStruct((n, 2 * nhid), _F32),
                jax.ShapeDtypeStruct((n, ncls), _F32),
                jax.ShapeDtypeStruct((ni, 1, nhid), _F32),
            ],
            compiler_params=par,
        )(mat, xcat, msk3, bt, w, wp, pa2)

    ha, u1, ra = gconv1(adj, W1, b1t, W2)
    hd, u2, rd = gconv1(diff, W3, b3t, W4)

    partial = pl.pallas_call(
        _gconv2a_kernel,
        grid=(ni,),
        in_specs=[
            pl.BlockSpec((bm, n), lambda i: (i, 0)),
            pl.BlockSpec((n, ncls), lambda i: (0, 0)),
        ],
        out_specs=pl.BlockSpec((bm, ncls), lambda i: (i, 0)),
        out_shape=jax.ShapeDtypeStruct((n, ncls), _F32),
        compiler_params=par,
    )(adj, u1)

    out, sc = pl.pallas_call(
        functools.partial(_gconv2b_kernel, nhid),
        grid=(ni,),
        in_specs=[
            pl.BlockSpec((bm, n), lambda i: (i, 0)),
            pl.BlockSpec((n, ncls), lambda i: (0, 0)),
            pl.BlockSpec((bm, ncls), lambda i: (i, 0)),
            pl.BlockSpec((bm, 2 * nhid), lambda i: (i, 0)),
            pl.BlockSpec((bm, 2 * nhid), lambda i: (i, 0)),
            pl.BlockSpec((ni, 1, nhid), lambda i: (0, 0, 0)),
            pl.BlockSpec((ni, 1, nhid), lambda i: (0, 0, 0)),
            pl.BlockSpec((nhid, nhid), lambda i: (0, 0)),
            pl.BlockSpec((1, ncls), lambda i: (0, 0)),
            pl.BlockSpec((1, 1), lambda i: (0, 0)),
            pl.BlockSpec((1, 1), lambda i: (0, 0)),
        ],
        out_specs=[
            pl.BlockSpec((bm, ncls), lambda i: (i, 0)),
            pl.BlockSpec((bm, 4), lambda i: (i, 0)),
        ],
        out_shape=[
            jax.ShapeDtypeStruct((n, ncls), _F32),
            jax.ShapeDtypeStruct((n, 4), _F32),
        ],
        compiler_params=par,
    )(diff, u2, partial, ha, hd, ra, rd, wb0, b24, bb2, inv)

    logits = sc.T.reshape(1, 4 * n)
    return (out, logits)


# speedup vs baseline: 1.0155x; 1.0155x over previous
import functools

import jax
import jax.numpy as jnp
from jax.experimental import pallas as pl
from jax.experimental.pallas import tpu as pltpu

_F32 = jnp.float32


def _blk(n, target):
    if n <= target:
        return n
    for b in range(target, 0, -1):
        if n % b == 0 and b % 8 == 0:
            return b
    return n


def _gconv1_kernel(nhid, nfeat, a_ref, x_ref, msk_ref, bt_ref, w_ref,
                   wp_ref, pa_ref, h_ref, u_ref, racc_ref):
    y = jnp.dot(a_ref[...], x_ref[...], preferred_element_type=_F32)
    w = w_ref[...]
    x = jnp.concatenate(
        [jnp.dot(y[:, :nfeat], w, preferred_element_type=_F32),
         jnp.dot(y[:, nfeat:], w, preferred_element_type=_F32)], axis=1)
    x = x + bt_ref[...]
    a = pa_ref[0, 0]
    h = jnp.where(x > 0, x, a * x)
    h_ref[...] = h
    u_ref[...] = jnp.dot(h[:, :nhid], wp_ref[...], preferred_element_type=_F32)
    m = msk_ref[0]
    racc_ref[...] = jnp.dot(m, h, preferred_element_type=_F32)[:, :nhid][None]


def _gconv2_kernel(nhid, a_ref, d_ref, u1_ref, u2_ref, ha_ref, hd_ref,
                   ra_ref, rd_ref, wb_ref, b24_ref, bb_ref, inv_ref,
                   out_ref, sc_ref):
    y = (jnp.dot(a_ref[...], u1_ref[...], preferred_element_type=_F32)
         + jnp.dot(d_ref[...], u2_ref[...], preferred_element_type=_F32)
         + b24_ref[...])
    mx = jnp.max(y, axis=1, keepdims=True)
    z = y - mx
    out_ref[...] = z - jnp.log(jnp.sum(jnp.exp(z), axis=1, keepdims=True))

    c1 = jax.nn.sigmoid(jnp.sum(ra_ref[...], axis=0) * inv_ref[0, 0])
    c2 = jax.nn.sigmoid(jnp.sum(rd_ref[...], axis=0) * inv_ref[0, 0])
    wb = wb_ref[...]
    dn = (((1,), (1,)), ((), ()))
    v1 = jax.lax.dot_general(wb, c1, dn, preferred_element_type=_F32)
    v2 = jax.lax.dot_general(wb, c2, dn, preferred_element_type=_F32)
    h1 = ha_ref[:, :nhid]
    h3 = ha_ref[:, nhid:]
    h2 = hd_ref[:, :nhid]
    h4 = hd_ref[:, nhid:]
    t1 = jnp.dot(h2, v1, preferred_element_type=_F32)
    t2 = jnp.dot(h1, v2, preferred_element_type=_F32)
    t3 = jnp.dot(h4, v1, preferred_element_type=_F32)
    t4 = jnp.dot(h3, v2, preferred_element_type=_F32)
    sc_ref[...] = jnp.concatenate([t1, t2, t3, t4], axis=1) + bb_ref[0, 0]


def kernel(feature, adj, diff, shuf_fts, sparse, msk, samp_bias1, samp_bias2,
           W1, b1, W2, b2, W3, b3, W4, b4, Wb, bb, prelu_a):
    del sparse, samp_bias1, samp_bias2
    n, nfeat = feature.shape
    nhid = W1.shape[1]
    ncls = W2.shape[1]

    bm = _blk(n, 400)
    ni = n // bm

    b1t = jnp.concatenate([b1, b1]).reshape(1, 2 * nhid)
    b3t = jnp.concatenate([b3, b3]).reshape(1, 2 * nhid)
    b24 = (b2 + b4).reshape(1, ncls)
    wb0 = Wb[0]
    bb2 = bb.reshape(1, 1)
    pa2 = prelu_a.reshape(1, 1)
    msk3 = msk.reshape(ni, 1, bm)
    inv = (1.0 / (n * jnp.sum(msk))).reshape(1, 1).astype(_F32)

    par = pltpu.CompilerParams(dimension_semantics=("parallel",),
                               vmem_limit_bytes=112 * 1024 * 1024)

    xcat = jnp.concatenate([feature, shuf_fts], axis=1)

    def gconv1(mat, w, bt, wp):
        return pl.pallas_call(
            functools.partial(_gconv1_kernel, nhid, nfeat),
            grid=(ni,),
            in_specs=[
                pl.BlockSpec((bm, n), lambda i: (i, 0)),
                pl.BlockSpec((n, 2 * nfeat), lambda i: (0, 0)),
                pl.BlockSpec((1, 1, bm), lambda i: (i, 0, 0)),
                pl.BlockSpec((1, 2 * nhid), lambda i: (0, 0)),
                pl.BlockSpec((nfeat, nhid), lambda i: (0, 0)),
                pl.BlockSpec((nhid, ncls), lambda i: (0, 0)),
                pl.BlockSpec((1, 1), lambda i: (0, 0)),
            ],
            out_specs=[
                pl.BlockSpec((bm, 2 * nhid), lambda i: (i, 0)),
                pl.BlockSpec((bm, ncls), lambda i: (i, 0)),
                pl.BlockSpec((1, 1, nhid), lambda i: (i, 0, 0)),
            ],
            out_shape=[
                jax.ShapeDtypeStruct((n, 2 * nhid), _F32),
                jax.ShapeDtypeStruct((n, ncls), _F32),
                jax.ShapeDtypeStruct((ni, 1, nhid), _F32),
            ],
            compiler_params=par,
        )(mat, xcat, msk3, bt, w, wp, pa2)

    ha, u1, ra = gconv1(adj, W1, b1t, W2)
    hd, u2, rd = gconv1(diff, W3, b3t, W4)

    bmc = bm // 2
    nic = n // bmc
    out, sc = pl.pallas_call(
        functools.partial(_gconv2_kernel, nhid),
        grid=(nic,),
        in_specs=[
            pl.BlockSpec((bmc, n), lambda i: (i, 0)),
            pl.BlockSpec((bmc, n), lambda i: (i, 0)),
            pl.BlockSpec((n, ncls), lambda i: (0, 0)),
            pl.BlockSpec((n, ncls), lambda i: (0, 0)),
            pl.BlockSpec((bmc, 2 * nhid), lambda i: (i, 0)),
            pl.BlockSpec((bmc, 2 * nhid), lambda i: (i, 0)),
            pl.BlockSpec((ni, 1, nhid), lambda i: (0, 0, 0)),
            pl.BlockSpec((ni, 1, nhid), lambda i: (0, 0, 0)),
            pl.BlockSpec((nhid, nhid), lambda i: (0, 0)),
            pl.BlockSpec((1, ncls), lambda i: (0, 0)),
            pl.BlockSpec((1, 1), lambda i: (0, 0)),
            pl.BlockSpec((1, 1), lambda i: (0, 0)),
        ],
        out_specs=[
            pl.BlockSpec((bmc, ncls), lambda i: (i, 0)),
            pl.BlockSpec((bmc, 4), lambda i: (i, 0)),
        ],
        out_shape=[
            jax.ShapeDtypeStruct((n, ncls), _F32),
            jax.ShapeDtypeStruct((n, 4), _F32),
        ],
        compiler_params=par,
    )(adj, diff, u1, u2, ha, hd, ra, rd, wb0, b24, bb2, inv)

    logits = sc.T.reshape(1, 4 * n)
    return (out, logits)
